# Initial kernel scaffold; baseline (speedup 1.0000x reference)
import jax
import jax.numpy as jnp
from jax.experimental import pallas as pl


def kernel(x1, edge_index1, x2, edge_index2, W1, b1, W_end, b_end, skip_connection):
    raise NotImplementedError("write your pallas kernel here")


# speedup vs baseline: 20.7911x; 20.7911x over previous
import functools

import jax
import jax.numpy as jnp
from jax import lax
from jax.experimental import pallas as pl
from jax.experimental.pallas import tpu as pltpu
from jax.experimental.pallas import tpu_sc as plsc

NC = 2
NS = 16
CH = 128


def _mm_body(x_ref, w_ref, o_ref):
    o_ref[...] = jnp.dot(x_ref[...], w_ref[...],
                         preferred_element_type=jnp.float32)


def _norm_body(h_ref, degp_ref, oh_ref, od_ref):
    deg = degp_ref[0] + degp_ref[1] + 1.0
    dinv = lax.rsqrt(deg)
    od_ref[...] = dinv
    oh_ref[...] = h_ref[...] * dinv


def _make_mid_body(n_real):
    def _mid_body(sp_ref, hp_ref, dinv_ref, b1_ref, w_ref, o_ref):
        agg = dinv_ref[...] * (sp_ref[0] + sp_ref[1] + hp_ref[...]) + b1_ref[...]
        h2 = jnp.maximum(agg, 0.0)
        z = jnp.dot(h2, w_ref[...], preferred_element_type=jnp.float32)
        zp = z * dinv_ref[...]
        rows = lax.broadcasted_iota(jnp.int32, zp.shape, 0)
        o_ref[...] = jnp.where(rows < n_real, zp, 0.0)
    return _mid_body


def _make_out_body(c_real):
    def _out_body(s2_ref, zp_ref, dinv_ref, be_ref, o_ref):
        agg = dinv_ref[...] * (s2_ref[0] + s2_ref[1] + zp_ref[...]) + be_ref[...]
        col = lax.broadcasted_iota(jnp.int32, agg.shape, 1)
        xm = jnp.where(col < c_real, agg, jnp.float32(-1e30))
        m = jnp.max(xm, axis=1, keepdims=True)
        ex = jnp.where(col < c_real, jnp.exp(agg - m), 0.0)
        lse = jnp.log(jnp.sum(ex, axis=1, keepdims=True)) + m
        o_ref[...] = agg - lse
    return _out_body


def _make_sc_kernels(n_pad, h_dim, k_chunks):
    rps = n_pad // NS
    mesh = plsc.VectorSubcoreMesh(core_axis_name="c", subcore_axis_name="s")
    out_t = jax.ShapeDtypeStruct((NC, n_pad, h_dim), jnp.float32)
    cparams = pltpu.CompilerParams(use_tc_tiling_on_sc=False)

    @functools.partial(
        pl.kernel, mesh=mesh, out_type=out_t, compiler_params=cparams,
        scratch_types=[
            pltpu.VMEM_SHARED((n_pad, h_dim), jnp.float32),
            pltpu.VMEM((CH,), jnp.int32),
            pltpu.VMEM((CH, h_dim), jnp.float32),
            pltpu.SemaphoreType.DMA,
        ])
    def deg_kernel(dst_hbm, ones_hbm, zeros_hbm, out_hbm, acc, idx_v, ones_v, sem):
        c = lax.axis_index("c")
        s = lax.axis_index("s")
        wid = c * NS + s
        pltpu.sync_copy(zeros_hbm.at[pl.ds(s * rps, rps)],
                        acc.at[pl.ds(s * rps, rps)])
        pltpu.sync_copy(ones_hbm, ones_v)
        plsc.subcore_barrier()

        def body(j, carry):
            pltpu.sync_copy(dst_hbm.at[wid * k_chunks + j], idx_v)
            pltpu.sync_copy(ones_v, acc.at[idx_v], add=True)
            return carry

        lax.fori_loop(0, k_chunks, body, 0)
        plsc.subcore_barrier()
        pltpu.sync_copy(acc.at[pl.ds(s * rps, rps)],
                        out_hbm.at[c, pl.ds(s * rps, rps)])

    @functools.partial(
        pl.kernel, mesh=mesh, out_type=out_t, compiler_params=cparams,
        scratch_types=[
            pltpu.VMEM_SHARED((n_pad, h_dim), jnp.float32),
            pltpu.VMEM((CH,), jnp.int32),
            pltpu.VMEM((CH,), jnp.int32),
            pltpu.VMEM((CH, h_dim), jnp.float32),
            pltpu.SemaphoreType.DMA,
        ])
    def gs_kernel(table_hbm, src_hbm, dst_hbm, zeros_hbm, out_hbm,
                  acc, sidx_v, didx_v, rows_v, sem):
        c = lax.axis_index("c")
        s = lax.axis_index("s")
        wid = c * NS + s
        pltpu.sync_copy(zeros_hbm.at[pl.ds(s * rps, rps)],
                        acc.at[pl.ds(s * rps, rps)])
        plsc.subcore_barrier()

        def body(j, carry):
            pltpu.sync_copy(src_hbm.at[wid * k_chunks + j], sidx_v)
            pltpu.sync_copy(dst_hbm.at[wid * k_chunks + j], didx_v)
            pltpu.async_copy(table_hbm.at[sidx_v], rows_v, sem).wait()
            pltpu.sync_copy(rows_v, acc.at[didx_v], add=True)
            return carry

        lax.fori_loop(0, k_chunks, body, 0)
        plsc.subcore_barrier()
        pltpu.sync_copy(acc.at[pl.ds(s * rps, rps)],
                        out_hbm.at[c, pl.ds(s * rps, rps)])

    return deg_kernel, gs_kernel


def kernel(x1, edge_index1, x2, edge_index2, W1, b1, W_end, b_end,
           skip_connection):
    del x1, edge_index1, skip_connection
    n, d = x2.shape
    h_dim = W1.shape[1]
    c_dim = W_end.shape[1]
    e = edge_index2.shape[1]
    f32 = jnp.float32

    n_pad = ((n + 1 + 127) // 128) * 128
    k_chunks = (e + NC * NS * CH - 1) // (NC * NS * CH)
    e_pad = NC * NS * CH * k_chunks

    src = edge_index2[0]
    dst = edge_index2[1]
    pad = e_pad - e
    srcp = jnp.concatenate([src, jnp.full((pad,), n, jnp.int32)]
                           ).reshape(NC * NS * k_chunks, CH)
    dstp = jnp.concatenate([dst, jnp.full((pad,), n, jnp.int32)]
                           ).reshape(NC * NS * k_chunks, CH)

    x2p = jnp.zeros((n_pad, d), f32).at[:n].set(x2)
    zeros_tab = jnp.zeros((n_pad, h_dim), f32)
    ones_rows = jnp.ones((CH, h_dim), f32)
    w_end_p = jnp.zeros((h_dim, h_dim), f32).at[:, :c_dim].set(W_end)
    b1_row = b1.reshape(1, h_dim)
    be_row = jnp.zeros((1, h_dim), f32).at[0, :c_dim].set(b_end)

    deg_kernel, gs_kernel = _make_sc_kernels(n_pad, h_dim, k_chunks)

    h = pl.pallas_call(
        _mm_body,
        out_shape=jax.ShapeDtypeStruct((n_pad, h_dim), f32))(x2p, W1)

    degp = deg_kernel(dstp, ones_rows, zeros_tab)

    hp, dinv = pl.pallas_call(
        _norm_body,
        out_shape=[jax.ShapeDtypeStruct((n_pad, h_dim), f32),
                   jax.ShapeDtypeStruct((n_pad, h_dim), f32)])(h, degp)

    sp = gs_kernel(hp, srcp, dstp, zeros_tab)

    zp = pl.pallas_call(
        _make_mid_body(n),
        out_shape=jax.ShapeDtypeStruct((n_pad, h_dim), f32))(
            sp, hp, dinv, b1_row, w_end_p)

    s2p = gs_kernel(zp, srcp, dstp, zeros_tab)

    outp = pl.pallas_call(
        _make_out_body(c_dim),
        out_shape=jax.ShapeDtypeStruct((n_pad, h_dim), f32))(
            s2p, zp, dinv, be_row)

    return outp[:n, :c_dim]


# speedup vs baseline: 35.9236x; 1.7278x over previous
import functools

import jax
import jax.numpy as jnp
from jax import lax
from jax.experimental import pallas as pl
from jax.experimental.pallas import tpu as pltpu
from jax.experimental.pallas import tpu_sc as plsc

NC = 2
NS = 16
CH = 128
GRP = 8


def _mm_body(x_ref, w_ref, o_ref):
    o_ref[...] = jnp.dot(x_ref[...], w_ref[...],
                         preferred_element_type=jnp.float32)


def _norm_body(h_ref, degp_ref, oh_ref, od_ref):
    deg = degp_ref[0] + degp_ref[1] + 1.0
    dinv = lax.rsqrt(deg)
    od_ref[...] = dinv
    oh_ref[...] = h_ref[...] * dinv


def _make_mid_body(n_real):
    def _mid_body(sp_ref, hp_ref, dinv_ref, b1_ref, w_ref, o_ref):
        agg = dinv_ref[...] * (sp_ref[0] + sp_ref[1] + hp_ref[...]) + b1_ref[...]
        h2 = jnp.maximum(agg, 0.0)
        z = jnp.dot(h2, w_ref[...], preferred_element_type=jnp.float32)
        zp = z * dinv_ref[...]
        rows = lax.broadcasted_iota(jnp.int32, zp.shape, 0)
        o_ref[...] = jnp.where(rows < n_real, zp, 0.0)
    return _mid_body


def _make_out_body(c_real):
    def _out_body(s2_ref, zp_ref, dinv_ref, be_ref, o_ref):
        agg = dinv_ref[...] * (s2_ref[0] + s2_ref[1] + zp_ref[...]) + be_ref[...]
        col = lax.broadcasted_iota(jnp.int32, agg.shape, 1)
        xm = jnp.where(col < c_real, agg, jnp.float32(-1e30))
        m = jnp.max(xm, axis=1, keepdims=True)
        ex = jnp.where(col < c_real, jnp.exp(agg - m), 0.0)
        lse = jnp.log(jnp.sum(ex, axis=1, keepdims=True)) + m
        o_ref[...] = agg - lse
    return _out_body


def _make_sc_kernels(n_pad, h_dim, k_chunks):
    rps = n_pad // NS
    mesh = plsc.VectorSubcoreMesh(core_axis_name="c", subcore_axis_name="s")
    out_t = jax.ShapeDtypeStruct((NC, n_pad, h_dim), jnp.float32)
    cparams = pltpu.CompilerParams(use_tc_tiling_on_sc=False)

    @functools.partial(
        pl.kernel, mesh=mesh, out_type=out_t, compiler_params=cparams,
        scratch_types=[
            pltpu.VMEM_SHARED((n_pad, h_dim), jnp.float32),
            pltpu.VMEM((k_chunks, CH), jnp.int32),
            pltpu.VMEM((CH, h_dim), jnp.float32),
            pltpu.SemaphoreType.DMA,
        ])
    def deg_kernel(dst_hbm, ones_hbm, zeros_hbm, out_hbm, acc, didx_v, ones_v, sem):
        c = lax.axis_index("c")
        s = lax.axis_index("s")
        wid = c * NS + s
        pltpu.sync_copy(zeros_hbm.at[pl.ds(s * rps, rps)],
                        acc.at[pl.ds(s * rps, rps)])
        pltpu.sync_copy(dst_hbm.at[wid], didx_v)
        pltpu.sync_copy(ones_hbm, ones_v)
        plsc.subcore_barrier()

        def body(j, carry):
            pltpu.sync_copy(ones_v, acc.at[didx_v.at[j]], add=True)
            return carry

        lax.fori_loop(0, k_chunks, body, 0)
        plsc.subcore_barrier()
        pltpu.sync_copy(acc.at[pl.ds(s * rps, rps)],
                        out_hbm.at[c, pl.ds(s * rps, rps)])

    @functools.partial(
        pl.kernel, mesh=mesh, out_type=out_t, compiler_params=cparams,
        scratch_types=[
            pltpu.VMEM_SHARED((n_pad, h_dim), jnp.float32),
            pltpu.VMEM((k_chunks, CH), jnp.int32),
            pltpu.VMEM((k_chunks, CH), jnp.int32),
            pltpu.VMEM((GRP, CH, h_dim), jnp.float32),
            pltpu.SemaphoreType.DMA,
        ])
    def gs_kernel(table_hbm, src_hbm, dst_hbm, zeros_hbm, out_hbm,
                  acc, sidx_v, didx_v, rows_v, sem):
        c = lax.axis_index("c")
        s = lax.axis_index("s")
        wid = c * NS + s
        pltpu.sync_copy(zeros_hbm.at[pl.ds(s * rps, rps)],
                        acc.at[pl.ds(s * rps, rps)])
        pltpu.sync_copy(src_hbm.at[wid], sidx_v)
        pltpu.sync_copy(dst_hbm.at[wid], didx_v)
        plsc.subcore_barrier()

        def body(g, carry):
            handles = [
                pltpu.async_copy(table_hbm.at[sidx_v.at[g * GRP + i]],
                                 rows_v.at[i], sem)
                for i in range(GRP)
            ]
            for h in handles:
                h.wait()
            for i in range(GRP):
                pltpu.sync_copy(rows_v.at[i], acc.at[didx_v.at[g * GRP + i]],
                                add=True)
            return carry

        lax.fori_loop(0, k_chunks // GRP, body, 0)
        plsc.subcore_barrier()
        pltpu.sync_copy(acc.at[pl.ds(s * rps, rps)],
                        out_hbm.at[c, pl.ds(s * rps, rps)])

    return deg_kernel, gs_kernel


def kernel(x1, edge_index1, x2, edge_index2, W1, b1, W_end, b_end,
           skip_connection):
    del x1, edge_index1, skip_connection
    n, d = x2.shape
    h_dim = W1.shape[1]
    c_dim = W_end.shape[1]
    e = edge_index2.shape[1]
    f32 = jnp.float32

    n_pad = ((n + 1 + 127) // 128) * 128
    k_chunks = (e + NC * NS * CH - 1) // (NC * NS * CH)
    k_chunks = ((k_chunks + GRP - 1) // GRP) * GRP
    e_pad = NC * NS * CH * k_chunks

    src = edge_index2[0]
    dst = edge_index2[1]
    pad = e_pad - e
    srcp = jnp.concatenate([src, jnp.full((pad,), n, jnp.int32)]
                           ).reshape(NC * NS, k_chunks, CH)
    dstp = jnp.concatenate([dst, jnp.full((pad,), n, jnp.int32)]
                           ).reshape(NC * NS, k_chunks, CH)

    x2p = jnp.zeros((n_pad, d), f32).at[:n].set(x2)
    zeros_tab = jnp.zeros((n_pad, h_dim), f32)
    ones_rows = jnp.ones((CH, h_dim), f32)
    w_end_p = jnp.zeros((h_dim, h_dim), f32).at[:, :c_dim].set(W_end)
    b1_row = b1.reshape(1, h_dim)
    be_row = jnp.zeros((1, h_dim), f32).at[0, :c_dim].set(b_end)

    deg_kernel, gs_kernel = _make_sc_kernels(n_pad, h_dim, k_chunks)

    h = pl.pallas_call(
        _mm_body,
        out_shape=jax.ShapeDtypeStruct((n_pad, h_dim), f32))(x2p, W1)

    degp = deg_kernel(dstp, ones_rows, zeros_tab)

    hp, dinv = pl.pallas_call(
        _norm_body,
        out_shape=[jax.ShapeDtypeStruct((n_pad, h_dim), f32),
                   jax.ShapeDtypeStruct((n_pad, h_dim), f32)])(h, degp)

    sp = gs_kernel(hp, srcp, dstp, zeros_tab)

    zp = pl.pallas_call(
        _make_mid_body(n),
        out_shape=jax.ShapeDtypeStruct((n_pad, h_dim), f32))(
            sp, hp, dinv, b1_row, w_end_p)

    s2p = gs_kernel(zp, srcp, dstp, zeros_tab)

    outp = pl.pallas_call(
        _make_out_body(c_dim),
        out_shape=jax.ShapeDtypeStruct((n_pad, h_dim), f32))(
            s2p, zp, dinv, be_row)

    return outp[:n, :c_dim]


# speedup vs baseline: 38.9191x; 1.0834x over previous
import functools

import jax
import jax.numpy as jnp
from jax import lax
from jax.experimental import pallas as pl
from jax.experimental.pallas import tpu as pltpu
from jax.experimental.pallas import tpu_sc as plsc

NC = 2
NS = 16
CH = 128
GRP = 8


def _mm_body(x_ref, w_ref, o_ref):
    o_ref[...] = jnp.dot(x_ref[...], w_ref[...],
                         preferred_element_type=jnp.float32)


def _norm_body(h_ref, degp_ref, oh_ref, od_ref):
    deg = degp_ref[0] + degp_ref[1] + 1.0
    dinv = lax.rsqrt(deg)
    od_ref[...] = dinv
    oh_ref[...] = h_ref[...] * dinv


def _make_mid_body(n_real):
    def _mid_body(sp_ref, hp_ref, dinv_ref, b1_ref, w_ref, o_ref):
        agg = dinv_ref[...] * (sp_ref[0] + sp_ref[1] + hp_ref[...]) + b1_ref[...]
        h2 = jnp.maximum(agg, 0.0)
        z = jnp.dot(h2, w_ref[...], preferred_element_type=jnp.float32)
        zp = z * dinv_ref[...]
        rows = lax.broadcasted_iota(jnp.int32, zp.shape, 0)
        o_ref[...] = jnp.where(rows < n_real, zp, 0.0)
    return _mid_body


def _make_out_body(c_real):
    def _out_body(s2_ref, zp_ref, dinv_ref, be_ref, o_ref):
        agg = dinv_ref[...] * (s2_ref[0] + s2_ref[1] + zp_ref[...]) + be_ref[...]
        col = lax.broadcasted_iota(jnp.int32, agg.shape, 1)
        xm = jnp.where(col < c_real, agg, jnp.float32(-1e30))
        m = jnp.max(xm, axis=1, keepdims=True)
        ex = jnp.where(col < c_real, jnp.exp(agg - m), 0.0)
        lse = jnp.log(jnp.sum(ex, axis=1, keepdims=True)) + m
        o_ref[...] = agg - lse
    return _out_body


def _make_sc_kernels(n_pad, h_dim, k_chunks):
    rps = n_pad // NS
    mesh = plsc.VectorSubcoreMesh(core_axis_name="c", subcore_axis_name="s")
    out_t = jax.ShapeDtypeStruct((NC, n_pad, h_dim), jnp.float32)
    cparams = pltpu.CompilerParams(use_tc_tiling_on_sc=False)

    @functools.partial(
        pl.kernel, mesh=mesh, out_type=out_t, compiler_params=cparams,
        scratch_types=[
            pltpu.VMEM_SHARED((n_pad, h_dim), jnp.float32),
            pltpu.VMEM((k_chunks, CH), jnp.int32),
            pltpu.VMEM((CH, h_dim), jnp.float32),
            pltpu.SemaphoreType.DMA,
        ])
    def deg_kernel(dst_hbm, ones_hbm, zeros_hbm, out_hbm, acc, didx_v, ones_v, sem):
        c = lax.axis_index("c")
        s = lax.axis_index("s")
        wid = c * NS + s
        pltpu.sync_copy(zeros_hbm.at[pl.ds(s * rps, rps)],
                        acc.at[pl.ds(s * rps, rps)])
        pltpu.sync_copy(dst_hbm.at[wid], didx_v)
        pltpu.sync_copy(ones_hbm, ones_v)
        plsc.subcore_barrier()

        def body(j, carry):
            pltpu.sync_copy(ones_v, acc.at[didx_v.at[j]], add=True)
            return carry

        lax.fori_loop(0, k_chunks, body, 0)
        plsc.subcore_barrier()
        pltpu.sync_copy(acc.at[pl.ds(s * rps, rps)],
                        out_hbm.at[c, pl.ds(s * rps, rps)])

    n_pairs = k_chunks // (2 * GRP)

    @functools.partial(
        pl.kernel, mesh=mesh, out_type=out_t, compiler_params=cparams,
        scratch_types=[
            pltpu.VMEM_SHARED((n_pad, h_dim), jnp.float32),
            pltpu.VMEM((k_chunks, CH), jnp.int32),
            pltpu.VMEM((k_chunks, CH), jnp.int32),
            pltpu.VMEM((GRP, CH, h_dim), jnp.float32),
            pltpu.VMEM((GRP, CH, h_dim), jnp.float32),
            pltpu.SemaphoreType.DMA,
            pltpu.SemaphoreType.DMA,
        ])
    def gs_kernel(table_hbm, src_hbm, dst_hbm, zeros_hbm, out_hbm,
                  acc, sidx_v, didx_v, rows_a, rows_b, sem_a, sem_b):
        c = lax.axis_index("c")
        s = lax.axis_index("s")
        wid = c * NS + s
        pltpu.sync_copy(zeros_hbm.at[pl.ds(s * rps, rps)],
                        acc.at[pl.ds(s * rps, rps)])
        pltpu.sync_copy(src_hbm.at[wid], sidx_v)
        pltpu.sync_copy(dst_hbm.at[wid], didx_v)
        plsc.subcore_barrier()

        def fire(g, buf, sem):
            for i in range(GRP):
                pltpu.async_copy(table_hbm.at[sidx_v.at[g * GRP + i]],
                                 buf.at[i], sem)

        def drain(g, buf, sem):
            for i in range(GRP):
                pltpu.make_async_copy(table_hbm.at[sidx_v.at[g * GRP + i]],
                                      buf.at[i], sem).wait()

        def scatter(g, buf):
            for i in range(GRP):
                pltpu.sync_copy(buf.at[i], acc.at[didx_v.at[g * GRP + i]],
                                add=True)

        fire(0, rows_a, sem_a)

        def body(t, carry):
            fire(2 * t + 1, rows_b, sem_b)
            drain(2 * t, rows_a, sem_a)
            scatter(2 * t, rows_a)

            @pl.when(t < n_pairs - 1)
            def _():
                fire(2 * t + 2, rows_a, sem_a)

            drain(2 * t + 1, rows_b, sem_b)
            scatter(2 * t + 1, rows_b)
            return carry

        lax.fori_loop(0, n_pairs, body, 0)
        plsc.subcore_barrier()
        pltpu.sync_copy(acc.at[pl.ds(s * rps, rps)],
                        out_hbm.at[c, pl.ds(s * rps, rps)])

    return deg_kernel, gs_kernel


def kernel(x1, edge_index1, x2, edge_index2, W1, b1, W_end, b_end,
           skip_connection):
    del x1, edge_index1, skip_connection
    n, d = x2.shape
    h_dim = W1.shape[1]
    c_dim = W_end.shape[1]
    e = edge_index2.shape[1]
    f32 = jnp.float32

    n_pad = ((n + 1 + 127) // 128) * 128
    k_chunks = (e + NC * NS * CH - 1) // (NC * NS * CH)
    k_chunks = ((k_chunks + 2 * GRP - 1) // (2 * GRP)) * (2 * GRP)
    e_pad = NC * NS * CH * k_chunks

    src = edge_index2[0]
    dst = edge_index2[1]
    pad = e_pad - e
    srcp = jnp.concatenate([src, jnp.full((pad,), n, jnp.int32)]
                           ).reshape(NC * NS, k_chunks, CH)
    dstp = jnp.concatenate([dst, jnp.full((pad,), n, jnp.int32)]
                           ).reshape(NC * NS, k_chunks, CH)

    x2p = jnp.zeros((n_pad, d), f32).at[:n].set(x2)
    zeros_tab = jnp.zeros((n_pad, h_dim), f32)
    ones_rows = jnp.ones((CH, h_dim), f32)
    w_end_p = jnp.zeros((h_dim, h_dim), f32).at[:, :c_dim].set(W_end)
    b1_row = b1.reshape(1, h_dim)
    be_row = jnp.zeros((1, h_dim), f32).at[0, :c_dim].set(b_end)

    deg_kernel, gs_kernel = _make_sc_kernels(n_pad, h_dim, k_chunks)

    h = pl.pallas_call(
        _mm_body,
        out_shape=jax.ShapeDtypeStruct((n_pad, h_dim), f32))(x2p, W1)

    degp = deg_kernel(dstp, ones_rows, zeros_tab)

    hp, dinv = pl.pallas_call(
        _norm_body,
        out_shape=[jax.ShapeDtypeStruct((n_pad, h_dim), f32),
                   jax.ShapeDtypeStruct((n_pad, h_dim), f32)])(h, degp)

    sp = gs_kernel(hp, srcp, dstp, zeros_tab)

    zp = pl.pallas_call(
        _make_mid_body(n),
        out_shape=jax.ShapeDtypeStruct((n_pad, h_dim), f32))(
            sp, hp, dinv, b1_row, w_end_p)

    s2p = gs_kernel(zp, srcp, dstp, zeros_tab)

    outp = pl.pallas_call(
        _make_out_body(c_dim),
        out_shape=jax.ShapeDtypeStruct((n_pad, h_dim), f32))(
            s2p, zp, dinv, be_row)

    return outp[:n, :c_dim]


# speedup vs baseline: 53.7076x; 1.3800x over previous
import functools

import jax
import jax.numpy as jnp
from jax import lax
from jax.experimental import pallas as pl
from jax.experimental.pallas import tpu as pltpu
from jax.experimental.pallas import tpu_sc as plsc

NC = 2
NS = 16
CH = 128
GRP = 8


def _mm_body(x_ref, w_ref, o_ref):
    o_ref[...] = jnp.dot(x_ref[...], w_ref[...],
                         preferred_element_type=jnp.float32)


def _norm_body(h_ref, degp_ref, oh_ref, od_ref):
    deg = degp_ref[0] + degp_ref[1] + 1.0
    dinv = lax.rsqrt(deg)
    od_ref[...] = dinv
    oh_ref[...] = h_ref[...] * dinv


def _make_mid_body(n_real):
    def _mid_body(sp_ref, hp_ref, dinv_ref, b1_ref, w_ref, o_ref):
        agg = dinv_ref[...] * (sp_ref[0] + sp_ref[1] + hp_ref[...]) + b1_ref[...]
        h2 = jnp.maximum(agg, 0.0)
        z = jnp.dot(h2, w_ref[...], preferred_element_type=jnp.float32)
        zp = z * dinv_ref[...]
        rows = lax.broadcasted_iota(jnp.int32, zp.shape, 0)
        o_ref[...] = jnp.where(rows < n_real, zp, 0.0)
    return _mid_body


def _make_out_body(c_real):
    def _out_body(s2_ref, zp_ref, dinv_ref, be_ref, o_ref):
        agg = dinv_ref[...] * (s2_ref[0] + s2_ref[1] + zp_ref[...]) + be_ref[...]
        col = lax.broadcasted_iota(jnp.int32, agg.shape, 1)
        xm = jnp.where(col < c_real, agg, jnp.float32(-1e30))
        m = jnp.max(xm, axis=1, keepdims=True)
        ex = jnp.where(col < c_real, jnp.exp(agg - m), 0.0)
        lse = jnp.log(jnp.sum(ex, axis=1, keepdims=True)) + m
        o_ref[...] = agg - lse
    return _out_body


def _make_sc_kernels(n_pad, h_dim, k_chunks):
    rps = n_pad // NS
    mesh = plsc.VectorSubcoreMesh(core_axis_name="c", subcore_axis_name="s")
    out_t = jax.ShapeDtypeStruct((NC, n_pad, h_dim), jnp.float32)
    cparams = pltpu.CompilerParams(use_tc_tiling_on_sc=False)

    @functools.partial(
        pl.kernel, mesh=mesh, out_type=out_t, compiler_params=cparams,
        scratch_types=[
            pltpu.VMEM_SHARED((n_pad, h_dim), jnp.float32),
            pltpu.VMEM((k_chunks, CH), jnp.int32),
            pltpu.VMEM((CH, h_dim), jnp.float32),
            pltpu.SemaphoreType.DMA,
        ])
    def deg_kernel(dst_hbm, ones_hbm, zeros_hbm, out_hbm, acc, didx_v, ones_v, sem):
        c = lax.axis_index("c")
        s = lax.axis_index("s")
        wid = c * NS + s
        pltpu.sync_copy(zeros_hbm.at[pl.ds(s * rps, rps)],
                        acc.at[pl.ds(s * rps, rps)])
        pltpu.sync_copy(dst_hbm.at[wid], didx_v)
        pltpu.sync_copy(ones_hbm, ones_v)
        plsc.subcore_barrier()

        def body(j, carry):
            pltpu.sync_copy(ones_v, acc.at[didx_v.at[j]], add=True)
            return carry

        lax.fori_loop(0, k_chunks, body, 0)
        plsc.subcore_barrier()
        pltpu.sync_copy(acc.at[pl.ds(s * rps, rps)],
                        out_hbm.at[c, pl.ds(s * rps, rps)])

    n_pairs = k_chunks // (2 * GRP)

    @functools.partial(
        pl.kernel, mesh=mesh, out_type=out_t, compiler_params=cparams,
        scratch_types=[
            pltpu.VMEM_SHARED((n_pad, h_dim), jnp.float32),
            pltpu.VMEM_SHARED((n_pad, h_dim), jnp.float32),
            pltpu.VMEM((k_chunks, CH), jnp.int32),
            pltpu.VMEM((k_chunks, CH), jnp.int32),
            pltpu.VMEM((GRP, CH, h_dim), jnp.float32),
            pltpu.VMEM((GRP, CH, h_dim), jnp.float32),
            pltpu.SemaphoreType.DMA,
            pltpu.SemaphoreType.DMA,
        ])
    def gs_kernel(table_hbm, src_hbm, dst_hbm, zeros_hbm, out_hbm,
                  acc, tab_s, sidx_v, didx_v, rows_a, rows_b, sem_a, sem_b):
        c = lax.axis_index("c")
        s = lax.axis_index("s")
        wid = c * NS + s
        pltpu.sync_copy(zeros_hbm.at[pl.ds(s * rps, rps)],
                        acc.at[pl.ds(s * rps, rps)])
        pltpu.sync_copy(table_hbm.at[pl.ds(s * rps, rps)],
                        tab_s.at[pl.ds(s * rps, rps)])
        pltpu.sync_copy(src_hbm.at[wid], sidx_v)
        pltpu.sync_copy(dst_hbm.at[wid], didx_v)
        plsc.subcore_barrier()

        def fire(g, buf, sem):
            for i in range(GRP):
                pltpu.async_copy(tab_s.at[sidx_v.at[g * GRP + i]],
                                 buf.at[i], sem)

        def drain(g, buf, sem):
            for i in range(GRP):
                pltpu.make_async_copy(tab_s.at[sidx_v.at[g * GRP + i]],
                                      buf.at[i], sem).wait()

        def scatter(g, buf):
            for i in range(GRP):
                pltpu.sync_copy(buf.at[i], acc.at[didx_v.at[g * GRP + i]],
                                add=True)

        fire(0, rows_a, sem_a)

        def body(t, carry):
            fire(2 * t + 1, rows_b, sem_b)
            drain(2 * t, rows_a, sem_a)
            scatter(2 * t, rows_a)

            @pl.when(t < n_pairs - 1)
            def _():
                fire(2 * t + 2, rows_a, sem_a)

            drain(2 * t + 1, rows_b, sem_b)
            scatter(2 * t + 1, rows_b)
            return carry

        lax.fori_loop(0, n_pairs, body, 0)
        plsc.subcore_barrier()
        pltpu.sync_copy(acc.at[pl.ds(s * rps, rps)],
                        out_hbm.at[c, pl.ds(s * rps, rps)])

    return deg_kernel, gs_kernel


def kernel(x1, edge_index1, x2, edge_index2, W1, b1, W_end, b_end,
           skip_connection):
    del x1, edge_index1, skip_connection
    n, d = x2.shape
    h_dim = W1.shape[1]
    c_dim = W_end.shape[1]
    e = edge_index2.shape[1]
    f32 = jnp.float32

    n_pad = ((n + 1 + 127) // 128) * 128
    k_chunks = (e + NC * NS * CH - 1) // (NC * NS * CH)
    k_chunks = ((k_chunks + 2 * GRP - 1) // (2 * GRP)) * (2 * GRP)
    e_pad = NC * NS * CH * k_chunks

    src = edge_index2[0]
    dst = edge_index2[1]
    pad = e_pad - e
    srcp = jnp.concatenate([src, jnp.full((pad,), n, jnp.int32)]
                           ).reshape(NC * NS, k_chunks, CH)
    dstp = jnp.concatenate([dst, jnp.full((pad,), n, jnp.int32)]
                           ).reshape(NC * NS, k_chunks, CH)

    x2p = jnp.zeros((n_pad, d), f32).at[:n].set(x2)
    zeros_tab = jnp.zeros((n_pad, h_dim), f32)
    ones_rows = jnp.ones((CH, h_dim), f32)
    w_end_p = jnp.zeros((h_dim, h_dim), f32).at[:, :c_dim].set(W_end)
    b1_row = b1.reshape(1, h_dim)
    be_row = jnp.zeros((1, h_dim), f32).at[0, :c_dim].set(b_end)

    deg_kernel, gs_kernel = _make_sc_kernels(n_pad, h_dim, k_chunks)

    h = pl.pallas_call(
        _mm_body,
        out_shape=jax.ShapeDtypeStruct((n_pad, h_dim), f32))(x2p, W1)

    degp = deg_kernel(dstp, ones_rows, zeros_tab)

    hp, dinv = pl.pallas_call(
        _norm_body,
        out_shape=[jax.ShapeDtypeStruct((n_pad, h_dim), f32),
                   jax.ShapeDtypeStruct((n_pad, h_dim), f32)])(h, degp)

    sp = gs_kernel(hp, srcp, dstp, zeros_tab)

    zp = pl.pallas_call(
        _make_mid_body(n),
        out_shape=jax.ShapeDtypeStruct((n_pad, h_dim), f32))(
            sp, hp, dinv, b1_row, w_end_p)

    s2p = gs_kernel(zp, srcp, dstp, zeros_tab)

    outp = pl.pallas_call(
        _make_out_body(c_dim),
        out_shape=jax.ShapeDtypeStruct((n_pad, h_dim), f32))(
            s2p, zp, dinv, be_row)

    return outp[:n, :c_dim]


# speedup vs baseline: 60.8456x; 1.1329x over previous
import functools

import jax
import jax.numpy as jnp
from jax import lax
from jax.experimental import pallas as pl
from jax.experimental.pallas import tpu as pltpu
from jax.experimental.pallas import tpu_sc as plsc

NC = 2
NS = 16
CH = 128
GRP = 8


def _mm_body(x_ref, w_ref, o_ref):
    o_ref[...] = jnp.dot(x_ref[...], w_ref[...],
                         preferred_element_type=jnp.float32)


def _make_out_body(c_real):
    def _out_body(s2_ref, g_ref, dinv_ref, w_ref, be_ref, o_ref):
        pre = dinv_ref[...] * (s2_ref[0] + s2_ref[1] + g_ref[...])
        agg = jnp.dot(pre, w_ref[...],
                      preferred_element_type=jnp.float32) + be_ref[...]
        col = lax.broadcasted_iota(jnp.int32, agg.shape, 1)
        xm = jnp.where(col < c_real, agg, jnp.float32(-1e30))
        m = jnp.max(xm, axis=1, keepdims=True)
        ex = jnp.where(col < c_real, jnp.exp(agg - m), 0.0)
        lse = jnp.log(jnp.sum(ex, axis=1, keepdims=True)) + m
        o_ref[...] = agg - lse
    return _out_body


def _newton_rsqrt(d):
    i = plsc.bitcast(d, jnp.int32)
    y = plsc.bitcast(jnp.int32(0x5F3759DF) - (i >> 1), jnp.float32)
    for _ in range(3):
        y = y * (1.5 - 0.5 * d * y * y)
    return y


def _make_sc_kernels(n_pad, h_dim, k_chunks, n_real):
    rps = n_pad // NS
    mesh = plsc.VectorSubcoreMesh(core_axis_name="c", subcore_axis_name="s")
    out_t = jax.ShapeDtypeStruct((NC, n_pad, h_dim), jnp.float32)
    cparams = pltpu.CompilerParams(use_tc_tiling_on_sc=False,
                                   needs_layout_passes=False)

    @functools.partial(
        pl.kernel, mesh=mesh, out_type=out_t, compiler_params=cparams,
        scratch_types=[
            pltpu.VMEM_SHARED((n_pad, h_dim), jnp.float32),
            pltpu.VMEM((k_chunks, CH), jnp.int32),
            pltpu.VMEM((CH, h_dim), jnp.float32),
            pltpu.SemaphoreType.DMA,
        ])
    def deg_kernel(dst_hbm, ones_hbm, zeros_hbm, out_hbm, acc, didx_v, ones_v, sem):
        c = lax.axis_index("c")
        s = lax.axis_index("s")
        wid = c * NS + s
        pltpu.sync_copy(zeros_hbm.at[pl.ds(s * rps, rps)],
                        acc.at[pl.ds(s * rps, rps)])
        pltpu.sync_copy(dst_hbm.at[wid], didx_v)
        pltpu.sync_copy(ones_hbm, ones_v)
        plsc.subcore_barrier()

        def body(j, carry):
            pltpu.sync_copy(ones_v, acc.at[didx_v.at[j]], add=True)
            return carry

        lax.fori_loop(0, k_chunks, body, 0)
        plsc.subcore_barrier()
        pltpu.sync_copy(acc.at[pl.ds(s * rps, rps)],
                        out_hbm.at[c, pl.ds(s * rps, rps)])

    n_pairs = k_chunks // (2 * GRP)

    def _pipeline(acc, tab_s, sidx_v, didx_v, rows_a, rows_b, sem_a, sem_b):
        def fire(g, buf, sem):
            for i in range(GRP):
                pltpu.async_copy(tab_s.at[sidx_v.at[g * GRP + i]],
                                 buf.at[i], sem)

        def drain(g, buf, sem):
            for i in range(GRP):
                pltpu.make_async_copy(tab_s.at[sidx_v.at[g * GRP + i]],
                                      buf.at[i], sem).wait()

        def scatter(g, buf):
            for i in range(GRP):
                pltpu.sync_copy(buf.at[i], acc.at[didx_v.at[g * GRP + i]],
                                add=True)

        fire(0, rows_a, sem_a)

        def body(t, carry):
            fire(2 * t + 1, rows_b, sem_b)
            drain(2 * t, rows_a, sem_a)
            scatter(2 * t, rows_a)

            @pl.when(t < n_pairs - 1)
            def _():
                fire(2 * t + 2, rows_a, sem_a)

            drain(2 * t + 1, rows_b, sem_b)
            scatter(2 * t + 1, rows_b)
            return carry

        lax.fori_loop(0, n_pairs, body, 0)

    gs_scratch = [
        pltpu.VMEM_SHARED((n_pad, h_dim), jnp.float32),
        pltpu.VMEM_SHARED((n_pad, h_dim), jnp.float32),
        pltpu.VMEM((k_chunks, CH), jnp.int32),
        pltpu.VMEM((k_chunks, CH), jnp.int32),
        pltpu.VMEM((GRP, CH, h_dim), jnp.float32),
        pltpu.VMEM((GRP, CH, h_dim), jnp.float32),
        pltpu.VMEM((rps, h_dim), jnp.float32),
        pltpu.VMEM((rps, h_dim), jnp.float32),
        pltpu.VMEM((rps, h_dim), jnp.float32),
        pltpu.VMEM((rps, h_dim), jnp.float32),
        pltpu.SemaphoreType.DMA,
        pltpu.SemaphoreType.DMA,
    ]

    @functools.partial(
        pl.kernel, mesh=mesh, compiler_params=cparams,
        out_type=[out_t,
                  jax.ShapeDtypeStruct((n_pad, h_dim), jnp.float32),
                  jax.ShapeDtypeStruct((n_pad, h_dim), jnp.float32)],
        scratch_types=gs_scratch)
    def gs1_kernel(h_hbm, degp_hbm, src_hbm, dst_hbm, zeros_hbm,
                   sp_out, hp_out, dinv_out,
                   acc, tab_s, sidx_v, didx_v, rows_a, rows_b,
                   h_v, d0_v, d1_v, dinv_v, sem_a, sem_b):
        c = lax.axis_index("c")
        s = lax.axis_index("s")
        wid = c * NS + s
        sl = pl.ds(s * rps, rps)
        pltpu.sync_copy(zeros_hbm.at[sl], acc.at[sl])
        pltpu.sync_copy(h_hbm.at[sl], h_v)
        pltpu.sync_copy(degp_hbm.at[0, sl], d0_v)
        pltpu.sync_copy(degp_hbm.at[1, sl], d1_v)
        pltpu.sync_copy(src_hbm.at[wid], sidx_v)
        pltpu.sync_copy(dst_hbm.at[wid], didx_v)

        def row(r, carry):
            d = d0_v[r] + d1_v[r] + 1.0
            y = _newton_rsqrt(d)
            dinv_v[r] = y
            h_v[r] = h_v[r] * y
            return carry

        lax.fori_loop(0, rps, row, 0)
        pltpu.sync_copy(h_v, tab_s.at[sl])

        @pl.when(c == 0)
        def _():
            pltpu.sync_copy(h_v, hp_out.at[sl])
            pltpu.sync_copy(dinv_v, dinv_out.at[sl])

        plsc.subcore_barrier()
        _pipeline(acc, tab_s, sidx_v, didx_v, rows_a, rows_b, sem_a, sem_b)
        plsc.subcore_barrier()
        pltpu.sync_copy(acc.at[sl], sp_out.at[c, sl])

    @functools.partial(
        pl.kernel, mesh=mesh, compiler_params=cparams,
        out_type=[out_t,
                  jax.ShapeDtypeStruct((n_pad, h_dim), jnp.float32)],
        scratch_types=gs_scratch + [pltpu.VMEM((1, h_dim), jnp.float32)])
    def gs2_kernel(sp_hbm, hp_hbm, dinv_hbm, b1_hbm, src_hbm, dst_hbm,
                   zeros_hbm, s2_out, g_out,
                   acc, tab_s, sidx_v, didx_v, rows_a, rows_b,
                   sp0_v, sp1_v, hp_v, dinv_v, sem_a, sem_b, b1_v):
        c = lax.axis_index("c")
        s = lax.axis_index("s")
        wid = c * NS + s
        sl = pl.ds(s * rps, rps)
        pltpu.sync_copy(zeros_hbm.at[sl], acc.at[sl])
        pltpu.sync_copy(sp_hbm.at[0, sl], sp0_v)
        pltpu.sync_copy(sp_hbm.at[1, sl], sp1_v)
        pltpu.sync_copy(hp_hbm.at[sl], hp_v)
        pltpu.sync_copy(dinv_hbm.at[sl], dinv_v)
        pltpu.sync_copy(b1_hbm, b1_v)
        pltpu.sync_copy(src_hbm.at[wid], sidx_v)
        pltpu.sync_copy(dst_hbm.at[wid], didx_v)

        def row(r, carry):
            y = dinv_v[r]
            a = y * (sp0_v[r] + sp1_v[r] + hp_v[r]) + b1_v[0]
            g = jnp.maximum(a, 0.0) * y
            hp_v[r] = jnp.where(s * rps + r < n_real, g, 0.0)
            return carry

        lax.fori_loop(0, rps, row, 0)
        pltpu.sync_copy(hp_v, tab_s.at[sl])

        @pl.when(c == 0)
        def _():
            pltpu.sync_copy(hp_v, g_out.at[sl])

        plsc.subcore_barrier()
        _pipeline(acc, tab_s, sidx_v, didx_v, rows_a, rows_b, sem_a, sem_b)
        plsc.subcore_barrier()
        pltpu.sync_copy(acc.at[sl], s2_out.at[c, sl])

    return deg_kernel, gs1_kernel, gs2_kernel


def kernel(x1, edge_index1, x2, edge_index2, W1, b1, W_end, b_end,
           skip_connection):
    del x1, edge_index1, skip_connection
    n, d = x2.shape
    h_dim = W1.shape[1]
    c_dim = W_end.shape[1]
    e = edge_index2.shape[1]
    f32 = jnp.float32

    n_pad = ((n + 1 + 127) // 128) * 128
    k_chunks = (e + NC * NS * CH - 1) // (NC * NS * CH)
    k_chunks = ((k_chunks + 2 * GRP - 1) // (2 * GRP)) * (2 * GRP)
    e_pad = NC * NS * CH * k_chunks

    src = edge_index2[0]
    dst = edge_index2[1]
    pad = e_pad - e
    srcp = jnp.concatenate([src, jnp.full((pad,), n, jnp.int32)]
                           ).reshape(NC * NS, k_chunks, CH)
    dstp = jnp.concatenate([dst, jnp.full((pad,), n, jnp.int32)]
                           ).reshape(NC * NS, k_chunks, CH)

    x2p = jnp.zeros((n_pad, d), f32).at[:n].set(x2)
    zeros_tab = jnp.zeros((n_pad, h_dim), f32)
    ones_rows = jnp.ones((CH, h_dim), f32)
    w_end_p = jnp.zeros((h_dim, h_dim), f32).at[:, :c_dim].set(W_end)
    b1_row = b1.reshape(1, h_dim)
    be_row = jnp.zeros((1, h_dim), f32).at[0, :c_dim].set(b_end)

    deg_kernel, gs1_kernel, gs2_kernel = _make_sc_kernels(
        n_pad, h_dim, k_chunks, n)

    h = pl.pallas_call(
        _mm_body,
        out_shape=jax.ShapeDtypeStruct((n_pad, h_dim), f32))(x2p, W1)

    degp = deg_kernel(dstp, ones_rows, zeros_tab)

    sp, hp, dinv = gs1_kernel(h, degp, srcp, dstp, zeros_tab)

    s2p, g = gs2_kernel(sp, hp, dinv, b1_row, srcp, dstp, zeros_tab)

    outp = pl.pallas_call(
        _make_out_body(c_dim),
        out_shape=jax.ShapeDtypeStruct((n_pad, h_dim), f32))(
            s2p, g, dinv, w_end_p, be_row)

    return outp[:n, :c_dim]
